# Initial kernel scaffold; baseline (speedup 1.0000x reference)
import jax
import jax.numpy as jnp
from jax.experimental import pallas as pl


def kernel(x, edge_index, batch, W1_l, W1_r, b1, W2_l, W2_r, b2, W3_l, W3_r, b3, Wlin1, blin1, Wlin2, blin2):
    raise NotImplementedError("write your pallas kernel here")


# speedup vs baseline: 3.1454x; 3.1454x over previous
import functools

import jax
import jax.numpy as jnp
from jax import lax
from jax.experimental import pallas as pl
from jax.experimental.pallas import tpu as pltpu
from jax.experimental.pallas import tpu_sc as plsc

F32 = jnp.float32
I32 = jnp.int32

NC = 2
NS = 16
NW = NC * NS

CH = 80
W128 = 128


def _layer_body(fin, h, s0_ref, s1_ref, c0_ref, c1_ref, hp_ref, wl_ref, wr_ref,
                b_ref, out_ref):
    cnt = c0_ref[:, :1] + c1_ref[:, :1]
    mean = (s0_ref[:, :fin] + s1_ref[:, :fin]) / jnp.maximum(cnt, 1.0)
    hv = hp_ref[:, :fin]
    o = (jnp.dot(mean, wl_ref[...], preferred_element_type=F32)
         + jnp.dot(hv, wr_ref[...], preferred_element_type=F32) + b_ref[...])
    out_ref[:, :h] = jnp.maximum(o, 0.0)
    out_ref[:, h:] = jnp.zeros((o.shape[0], W128 - h), F32)


def _layer_call(s0, s1, c0, c1, hp, wl, wr, b, block_n=512):
    n = s0.shape[0]
    fin, h = wl.shape
    grid = (pl.cdiv(n, block_n),)
    return pl.pallas_call(
        functools.partial(_layer_body, fin, h),
        grid=grid,
        in_specs=[
            pl.BlockSpec((block_n, W128), lambda i: (i, 0)),
            pl.BlockSpec((block_n, W128), lambda i: (i, 0)),
            pl.BlockSpec((block_n, W128), lambda i: (i, 0)),
            pl.BlockSpec((block_n, W128), lambda i: (i, 0)),
            pl.BlockSpec((block_n, W128), lambda i: (i, 0)),
            pl.BlockSpec((fin, h), lambda i: (0, 0)),
            pl.BlockSpec((fin, h), lambda i: (0, 0)),
            pl.BlockSpec((1, h), lambda i: (0, 0)),
        ],
        out_specs=pl.BlockSpec((block_n, W128), lambda i: (i, 0)),
        out_shape=jax.ShapeDtypeStruct((n, W128), F32),
    )(s0, s1, c0, c1, hp, wl, wr, b)


def _rank_body(n, br_blk, bc_blk, keyr_ref, batr_ref, keyc_ref, batc_ref,
               rank_ref):
    i = pl.program_id(0)
    j = pl.program_id(1)

    @pl.when(j == 0)
    def _():
        rank_ref[...] = jnp.zeros_like(rank_ref)

    rowid = i * br_blk + lax.broadcasted_iota(I32, (br_blk, 1), 0)
    colid = j * bc_blk + lax.broadcasted_iota(I32, (1, bc_blk), 1)
    rvalid = rowid < n
    cvalid = colid < n
    br = batr_ref[...]
    bc = batc_ref[...]
    bmin_r = jnp.min(jnp.where(rvalid, br, 2147483647))
    bmax_r = jnp.max(jnp.where(rvalid, br, -1))
    bmin_c = jnp.min(jnp.where(cvalid, bc, 2147483647))
    bmax_c = jnp.max(jnp.where(cvalid, bc, -1))

    @pl.when((bmin_r <= bmax_c) & (bmin_c <= bmax_r))
    def _():
        kr = keyr_ref[...]
        kc = keyc_ref[...]
        earlier = (kc > kr) | ((kc == kr) & (colid < rowid))
        cond = (bc == br) & cvalid & earlier
        rank_ref[...] += jnp.sum(cond.astype(I32), axis=1, keepdims=True)


def _rank_call(keyr, batr, keyc, batc, block=512):
    n = keyr.shape[0]
    grid = (pl.cdiv(n, block), pl.cdiv(n, block))
    return pl.pallas_call(
        functools.partial(_rank_body, n, block, block),
        grid=grid,
        in_specs=[
            pl.BlockSpec((block, 1), lambda i, j: (i, 0)),
            pl.BlockSpec((block, 1), lambda i, j: (i, 0)),
            pl.BlockSpec((1, block), lambda i, j: (0, j)),
            pl.BlockSpec((1, block), lambda i, j: (0, j)),
        ],
        out_specs=pl.BlockSpec((block, 1), lambda i, j: (i, 0)),
        out_shape=jax.ShapeDtypeStruct((n, 1), I32),
    )(keyr, batr, keyc, batc)


def _mlp_body(x_ref, w1_ref, b1_ref, w2_ref, b2_ref, out_ref):
    hv = jnp.maximum(
        jnp.dot(x_ref[...], w1_ref[...], preferred_element_type=F32)
        + b1_ref[...], 0.0)
    o = jnp.dot(hv, w2_ref[...], preferred_element_type=F32) + b2_ref[...]
    m = jnp.max(o, axis=-1, keepdims=True)
    e = jnp.exp(o - m)
    out_ref[...] = (o - m) - jnp.log(jnp.sum(e, axis=-1, keepdims=True))


def _mlp_call(x, w1, b1, w2, b2):
    g, d = x.shape
    h = w1.shape[1]
    c = w2.shape[1]
    return pl.pallas_call(
        _mlp_body,
        in_specs=[
            pl.BlockSpec((g, d), lambda: (0, 0)),
            pl.BlockSpec((d, h), lambda: (0, 0)),
            pl.BlockSpec((1, h), lambda: (0, 0)),
            pl.BlockSpec((h, c), lambda: (0, 0)),
            pl.BlockSpec((1, c), lambda: (0, 0)),
        ],
        out_specs=pl.BlockSpec((g, c), lambda: (0, 0)),
        out_shape=jax.ShapeDtypeStruct((g, c), F32),
    )(x, w1, b1, w2, b2)


def _make_agg_kernel(n, e):
    nchunks = e // CH
    rps = (n // NS) & ~7
    tail = n - NS * rps
    mesh = plsc.VectorSubcoreMesh(core_axis_name="c", subcore_axis_name="s")

    @functools.partial(
        pl.kernel,
        out_type=jax.ShapeDtypeStruct((NC * n, W128), F32),
        mesh=mesh,
        scratch_types=[
            pltpu.VMEM((CH,), I32),
            pltpu.VMEM((CH,), I32),
            pltpu.VMEM((CH, W128), F32),
            pltpu.VMEM_SHARED((n, W128), F32),
            pltpu.SemaphoreType.DMA,
        ],
    )
    def agg(p_hbm, src_hbm, dst_hbm, zeros_hbm, out_hbm,
            sidx_v, didx_v, rows_v, acc_sh, sem):
        c = lax.axis_index("c")
        s = lax.axis_index("s")
        w = s * NC + c
        pltpu.sync_copy(zeros_hbm.at[pl.ds(s * rps, rps)],
                        acc_sh.at[pl.ds(s * rps, rps)])
        if tail:
            @pl.when(s == 0)
            def _():
                pltpu.sync_copy(zeros_hbm.at[pl.ds(NS * rps, tail)],
                                acc_sh.at[pl.ds(NS * rps, tail)])
        plsc.subcore_barrier()

        def body(j, carry):
            off = (w + NW * j) * CH
            pltpu.sync_copy(src_hbm.at[pl.ds(off, CH)], sidx_v)
            pltpu.sync_copy(dst_hbm.at[pl.ds(off, CH)], didx_v)
            pltpu.async_copy(p_hbm.at[sidx_v], rows_v, sem).wait()
            pltpu.sync_copy(rows_v, acc_sh.at[didx_v], add=True)
            return carry

        lax.fori_loop(0, nchunks // NW, body, 0)
        plsc.subcore_barrier()
        pltpu.sync_copy(acc_sh.at[pl.ds(s * rps, rps)],
                        out_hbm.at[pl.ds(c * n + s * rps, rps)])
        if tail:
            @pl.when(s == 0)
            def _():
                pltpu.sync_copy(acc_sh.at[pl.ds(NS * rps, tail)],
                                out_hbm.at[pl.ds(c * n + NS * rps, tail)])

    return agg


def _make_cnt_kernel(n, e):
    nchunks = e // CH
    rps = (n // NS) & ~7
    tail = n - NS * rps
    mesh = plsc.VectorSubcoreMesh(core_axis_name="c", subcore_axis_name="s")

    @functools.partial(
        pl.kernel,
        out_type=jax.ShapeDtypeStruct((NC * n, W128), F32),
        mesh=mesh,
        scratch_types=[
            pltpu.VMEM((CH,), I32),
            pltpu.VMEM((CH, W128), F32),
            pltpu.VMEM_SHARED((n, W128), F32),
        ],
    )
    def cnt(dst_hbm, ones_hbm, zeros_hbm, out_hbm, didx_v, ones_v, acc_sh):
        c = lax.axis_index("c")
        s = lax.axis_index("s")
        w = s * NC + c
        pltpu.sync_copy(ones_hbm, ones_v)
        pltpu.sync_copy(zeros_hbm.at[pl.ds(s * rps, rps)],
                        acc_sh.at[pl.ds(s * rps, rps)])
        if tail:
            @pl.when(s == 0)
            def _():
                pltpu.sync_copy(zeros_hbm.at[pl.ds(NS * rps, tail)],
                                acc_sh.at[pl.ds(NS * rps, tail)])
        plsc.subcore_barrier()

        def body(j, carry):
            off = (w + NW * j) * CH
            pltpu.sync_copy(dst_hbm.at[pl.ds(off, CH)], didx_v)
            pltpu.sync_copy(ones_v, acc_sh.at[didx_v], add=True)
            return carry

        lax.fori_loop(0, nchunks // NW, body, 0)
        plsc.subcore_barrier()
        pltpu.sync_copy(acc_sh.at[pl.ds(s * rps, rps)],
                        out_hbm.at[pl.ds(c * n + s * rps, rps)])
        if tail:
            @pl.when(s == 0)
            def _():
                pltpu.sync_copy(acc_sh.at[pl.ds(NS * rps, tail)],
                                out_hbm.at[pl.ds(c * n + NS * rps, tail)])

    return cnt


def _make_pool_kernel(n, num_graphs, k):
    nvalid = num_graphs * k
    out_rows = nvalid + NS
    rows_per_sub = -(-out_rows // (NS * 8)) * 8
    out_rows = rows_per_sub * NS
    nchunks = n // CH
    mesh = plsc.VectorSubcoreMesh(core_axis_name="c", subcore_axis_name="s",
                                  num_cores=1)

    @functools.partial(
        pl.kernel,
        out_type=jax.ShapeDtypeStruct((out_rows, W128), F32),
        mesh=mesh,
        scratch_types=[
            pltpu.VMEM((CH,), I32),
            pltpu.VMEM((CH,), I32),
            pltpu.VMEM((CH,), I32),
            pltpu.VMEM((CH, W128), F32),
            pltpu.SemaphoreType.DMA,
        ],
    )
    def pool(h_hbm, rank_hbm, batch_hbm, zeros_hbm, out_hbm,
             rank_v, bat_v, slot_v, rows_v, sem):
        s = lax.axis_index("s")
        pltpu.sync_copy(zeros_hbm.at[pl.ds(s * rows_per_sub, rows_per_sub)],
                        out_hbm.at[pl.ds(s * rows_per_sub, rows_per_sub)])
        plsc.subcore_barrier()

        def body(j, carry):
            chunk = s + NS * j

            @pl.when(chunk < nchunks)
            def _():
                off = chunk * CH
                pltpu.sync_copy(rank_hbm.at[pl.ds(off, CH)], rank_v)
                pltpu.sync_copy(batch_hbm.at[pl.ds(off, CH)], bat_v)
                pltpu.sync_copy(h_hbm.at[pl.ds(off, CH)], rows_v)
                for v in range(CH // 16):
                    rv = rank_v[pl.ds(v * 16, 16)]
                    bv = bat_v[pl.ds(v * 16, 16)]
                    slot_v[pl.ds(v * 16, 16)] = jnp.where(
                        rv < k, bv * k + rv, nvalid + s)
                pltpu.async_copy(rows_v, out_hbm.at[slot_v], sem).wait()
            return carry

        lax.fori_loop(0, -(-nchunks // NS), body, 0)

    return pool


def kernel(x, edge_index, batch, W1_l, W1_r, b1, W2_l, W2_r, b2,
           W3_l, W3_r, b3, Wlin1, blin1, Wlin2, blin2):
    n, fin = x.shape
    e = edge_index.shape[1]
    h = W1_l.shape[1]
    num_graphs, k = 100, 30

    src = edge_index[0]
    dst = edge_index[1]
    zeros128 = jnp.zeros((n, W128), F32)
    ones128 = jnp.ones((CH, W128), F32)

    agg = _make_agg_kernel(n, e)
    cntk = _make_cnt_kernel(n, e)

    cnt_parts = cntk(dst, ones128, zeros128)
    c0, c1 = cnt_parts[:n], cnt_parts[n:]

    s_parts = agg(x, src, dst, zeros128)
    hcur = _layer_call(s_parts[:n], s_parts[n:], c0, c1, x,
                       W1_l, W1_r, b1.reshape(1, -1))
    s_parts = agg(hcur, src, dst, zeros128)
    hcur = _layer_call(s_parts[:n], s_parts[n:], c0, c1, hcur,
                       W2_l, W2_r, b2.reshape(1, -1))
    s_parts = agg(hcur, src, dst, zeros128)
    h3 = _layer_call(s_parts[:n], s_parts[n:], c0, c1, hcur,
                     W3_l, W3_r, b3.reshape(1, -1))

    key_col = h3[:, h - 1:h]
    rank = _rank_call(key_col, batch.reshape(n, 1),
                      key_col.reshape(1, n), batch.reshape(1, n))

    poolk = _make_pool_kernel(n, num_graphs, k)
    pooled_buf = poolk(h3, rank.reshape(n), batch, zeros128)
    pooled = pooled_buf[:num_graphs * k, :h].reshape(num_graphs, k * h)

    return _mlp_call(pooled, Wlin1, blin1.reshape(1, -1),
                     Wlin2, blin2.reshape(1, -1))


# speedup vs baseline: 4.1727x; 1.3266x over previous
import functools

import jax
import jax.numpy as jnp
from jax import lax
from jax.experimental import pallas as pl
from jax.experimental.pallas import tpu as pltpu
from jax.experimental.pallas import tpu_sc as plsc

F32 = jnp.float32
I32 = jnp.int32

NC = 2
NS = 16
NW = NC * NS

CH = 80
W128 = 128


def _layer_body(fin, h, s0_ref, s1_ref, c0_ref, c1_ref, hp_ref, wl_ref, wr_ref,
                b_ref, out_ref):
    cnt = c0_ref[:, :1] + c1_ref[:, :1]
    mean = (s0_ref[:, :fin] + s1_ref[:, :fin]) / jnp.maximum(cnt, 1.0)
    hv = hp_ref[:, :fin]
    o = (jnp.dot(mean, wl_ref[...], preferred_element_type=F32)
         + jnp.dot(hv, wr_ref[...], preferred_element_type=F32) + b_ref[...])
    out_ref[:, :h] = jnp.maximum(o, 0.0)
    out_ref[:, h:] = jnp.zeros((o.shape[0], W128 - h), F32)


def _layer_call(s0, s1, c0, c1, hp, wl, wr, b, block_n=512):
    n = s0.shape[0]
    fin, h = wl.shape
    grid = (pl.cdiv(n, block_n),)
    return pl.pallas_call(
        functools.partial(_layer_body, fin, h),
        grid=grid,
        in_specs=[
            pl.BlockSpec((block_n, W128), lambda i: (i, 0)),
            pl.BlockSpec((block_n, W128), lambda i: (i, 0)),
            pl.BlockSpec((block_n, W128), lambda i: (i, 0)),
            pl.BlockSpec((block_n, W128), lambda i: (i, 0)),
            pl.BlockSpec((block_n, W128), lambda i: (i, 0)),
            pl.BlockSpec((fin, h), lambda i: (0, 0)),
            pl.BlockSpec((fin, h), lambda i: (0, 0)),
            pl.BlockSpec((1, h), lambda i: (0, 0)),
        ],
        out_specs=pl.BlockSpec((block_n, W128), lambda i: (i, 0)),
        out_shape=jax.ShapeDtypeStruct((n, W128), F32),
    )(s0, s1, c0, c1, hp, wl, wr, b)


def _rank_body(n, br_blk, bc_blk, keyr_ref, batr_ref, keyc_ref, batc_ref,
               rank_ref):
    i = pl.program_id(0)
    j = pl.program_id(1)

    @pl.when(j == 0)
    def _():
        rank_ref[...] = jnp.zeros_like(rank_ref)

    rowid = i * br_blk + lax.broadcasted_iota(I32, (br_blk, 1), 0)
    colid = j * bc_blk + lax.broadcasted_iota(I32, (1, bc_blk), 1)
    rvalid = rowid < n
    cvalid = colid < n
    br = batr_ref[...]
    bc = batc_ref[...]
    bmin_r = jnp.min(jnp.where(rvalid, br, 2147483647))
    bmax_r = jnp.max(jnp.where(rvalid, br, -1))
    bmin_c = jnp.min(jnp.where(cvalid, bc, 2147483647))
    bmax_c = jnp.max(jnp.where(cvalid, bc, -1))

    @pl.when((bmin_r <= bmax_c) & (bmin_c <= bmax_r))
    def _():
        kr = keyr_ref[...]
        kc = keyc_ref[...]
        earlier = (kc > kr) | ((kc == kr) & (colid < rowid))
        cond = (bc == br) & cvalid & earlier
        rank_ref[...] += jnp.sum(cond.astype(I32), axis=1, keepdims=True)


def _rank_call(keyr, batr, keyc, batc, block=512):
    n = keyr.shape[0]
    grid = (pl.cdiv(n, block), pl.cdiv(n, block))
    return pl.pallas_call(
        functools.partial(_rank_body, n, block, block),
        grid=grid,
        in_specs=[
            pl.BlockSpec((block, 1), lambda i, j: (i, 0)),
            pl.BlockSpec((block, 1), lambda i, j: (i, 0)),
            pl.BlockSpec((1, block), lambda i, j: (0, j)),
            pl.BlockSpec((1, block), lambda i, j: (0, j)),
        ],
        out_specs=pl.BlockSpec((block, 1), lambda i, j: (i, 0)),
        out_shape=jax.ShapeDtypeStruct((n, 1), I32),
    )(keyr, batr, keyc, batc)


def _mlp_body(x_ref, w1_ref, b1_ref, w2_ref, b2_ref, out_ref):
    hv = jnp.maximum(
        jnp.dot(x_ref[...], w1_ref[...], preferred_element_type=F32)
        + b1_ref[...], 0.0)
    o = jnp.dot(hv, w2_ref[...], preferred_element_type=F32) + b2_ref[...]
    m = jnp.max(o, axis=-1, keepdims=True)
    e = jnp.exp(o - m)
    out_ref[...] = (o - m) - jnp.log(jnp.sum(e, axis=-1, keepdims=True))


def _mlp_call(x, w1, b1, w2, b2):
    g, d = x.shape
    h = w1.shape[1]
    c = w2.shape[1]
    return pl.pallas_call(
        _mlp_body,
        in_specs=[
            pl.BlockSpec((g, d), lambda: (0, 0)),
            pl.BlockSpec((d, h), lambda: (0, 0)),
            pl.BlockSpec((1, h), lambda: (0, 0)),
            pl.BlockSpec((h, c), lambda: (0, 0)),
            pl.BlockSpec((1, c), lambda: (0, 0)),
        ],
        out_specs=pl.BlockSpec((g, c), lambda: (0, 0)),
        out_shape=jax.ShapeDtypeStruct((g, c), F32),
    )(x, w1, b1, w2, b2)


def _make_agg_kernel(n, e):
    nchunks = e // CH
    rps = (n // NS) & ~7
    tail = n - NS * rps
    mesh = plsc.VectorSubcoreMesh(core_axis_name="c", subcore_axis_name="s")

    iters = nchunks // NW
    assert iters % 2 == 1 and iters >= 3

    @functools.partial(
        pl.kernel,
        out_type=jax.ShapeDtypeStruct((NC * n, W128), F32),
        mesh=mesh,
        scratch_types=[
            pltpu.VMEM((CH,), I32),
            pltpu.VMEM((CH,), I32),
            pltpu.VMEM((CH,), I32),
            pltpu.VMEM((CH,), I32),
            pltpu.VMEM((CH, W128), F32),
            pltpu.VMEM((CH, W128), F32),
            pltpu.VMEM_SHARED((n, W128), F32),
            pltpu.SemaphoreType.DMA,
            pltpu.SemaphoreType.DMA,
        ],
    )
    def agg(p_hbm, src_hbm, dst_hbm, zeros_hbm, out_hbm,
            sidx_a, didx_a, sidx_b, didx_b, rows_a, rows_b, acc_sh,
            sem_a, sem_b):
        c = lax.axis_index("c")
        s = lax.axis_index("s")
        w = s * NC + c
        pltpu.sync_copy(zeros_hbm.at[pl.ds(s * rps, rps)],
                        acc_sh.at[pl.ds(s * rps, rps)])
        if tail:
            @pl.when(s == 0)
            def _():
                pltpu.sync_copy(zeros_hbm.at[pl.ds(NS * rps, tail)],
                                acc_sh.at[pl.ds(NS * rps, tail)])
        plsc.subcore_barrier()

        def off(j):
            return (w + NW * j) * CH

        pltpu.sync_copy(src_hbm.at[pl.ds(off(0), CH)], sidx_a)
        pltpu.sync_copy(dst_hbm.at[pl.ds(off(0), CH)], didx_a)
        pltpu.async_copy(p_hbm.at[sidx_a], rows_a, sem_a)

        def body(g, carry):
            ca = 2 * g
            cb = ca + 1
            cn = ca + 2
            pltpu.sync_copy(src_hbm.at[pl.ds(off(cb), CH)], sidx_b)
            pltpu.sync_copy(dst_hbm.at[pl.ds(off(cb), CH)], didx_b)
            pltpu.async_copy(p_hbm.at[sidx_b], rows_b, sem_b)
            pltpu.make_async_copy(p_hbm.at[sidx_a], rows_a, sem_a).wait()
            pltpu.sync_copy(rows_a, acc_sh.at[didx_a], add=True)
            pltpu.sync_copy(src_hbm.at[pl.ds(off(cn), CH)], sidx_a)
            pltpu.sync_copy(dst_hbm.at[pl.ds(off(cn), CH)], didx_a)
            pltpu.async_copy(p_hbm.at[sidx_a], rows_a, sem_a)
            pltpu.make_async_copy(p_hbm.at[sidx_b], rows_b, sem_b).wait()
            pltpu.sync_copy(rows_b, acc_sh.at[didx_b], add=True)
            return carry

        lax.fori_loop(0, (iters - 1) // 2, body, 0)
        pltpu.make_async_copy(p_hbm.at[sidx_a], rows_a, sem_a).wait()
        pltpu.sync_copy(rows_a, acc_sh.at[didx_a], add=True)
        plsc.subcore_barrier()
        pltpu.sync_copy(acc_sh.at[pl.ds(s * rps, rps)],
                        out_hbm.at[pl.ds(c * n + s * rps, rps)])
        if tail:
            @pl.when(s == 0)
            def _():
                pltpu.sync_copy(acc_sh.at[pl.ds(NS * rps, tail)],
                                out_hbm.at[pl.ds(c * n + NS * rps, tail)])

    return agg


def _make_cnt_kernel(n, e):
    nchunks = e // CH
    rps = (n // NS) & ~7
    tail = n - NS * rps
    mesh = plsc.VectorSubcoreMesh(core_axis_name="c", subcore_axis_name="s")

    @functools.partial(
        pl.kernel,
        out_type=jax.ShapeDtypeStruct((NC * n, W128), F32),
        mesh=mesh,
        scratch_types=[
            pltpu.VMEM((CH,), I32),
            pltpu.VMEM((CH, W128), F32),
            pltpu.VMEM_SHARED((n, W128), F32),
        ],
    )
    def cnt(dst_hbm, ones_hbm, zeros_hbm, out_hbm, didx_v, ones_v, acc_sh):
        c = lax.axis_index("c")
        s = lax.axis_index("s")
        w = s * NC + c
        pltpu.sync_copy(ones_hbm, ones_v)
        pltpu.sync_copy(zeros_hbm.at[pl.ds(s * rps, rps)],
                        acc_sh.at[pl.ds(s * rps, rps)])
        if tail:
            @pl.when(s == 0)
            def _():
                pltpu.sync_copy(zeros_hbm.at[pl.ds(NS * rps, tail)],
                                acc_sh.at[pl.ds(NS * rps, tail)])
        plsc.subcore_barrier()

        def body(j, carry):
            off = (w + NW * j) * CH
            pltpu.sync_copy(dst_hbm.at[pl.ds(off, CH)], didx_v)
            pltpu.sync_copy(ones_v, acc_sh.at[didx_v], add=True)
            return carry

        lax.fori_loop(0, nchunks // NW, body, 0)
        plsc.subcore_barrier()
        pltpu.sync_copy(acc_sh.at[pl.ds(s * rps, rps)],
                        out_hbm.at[pl.ds(c * n + s * rps, rps)])
        if tail:
            @pl.when(s == 0)
            def _():
                pltpu.sync_copy(acc_sh.at[pl.ds(NS * rps, tail)],
                                out_hbm.at[pl.ds(c * n + NS * rps, tail)])

    return cnt


def _make_pool_kernel(n, num_graphs, k):
    nvalid = num_graphs * k
    out_rows = nvalid + NS
    rows_per_sub = -(-out_rows // (NS * 8)) * 8
    out_rows = rows_per_sub * NS
    nchunks = n // CH
    mesh = plsc.VectorSubcoreMesh(core_axis_name="c", subcore_axis_name="s",
                                  num_cores=1)

    @functools.partial(
        pl.kernel,
        out_type=jax.ShapeDtypeStruct((out_rows, W128), F32),
        mesh=mesh,
        scratch_types=[
            pltpu.VMEM((CH,), I32),
            pltpu.VMEM((CH,), I32),
            pltpu.VMEM((CH,), I32),
            pltpu.VMEM((CH, W128), F32),
            pltpu.SemaphoreType.DMA,
        ],
    )
    def pool(h_hbm, rank_hbm, batch_hbm, zeros_hbm, out_hbm,
             rank_v, bat_v, slot_v, rows_v, sem):
        s = lax.axis_index("s")
        pltpu.sync_copy(zeros_hbm.at[pl.ds(s * rows_per_sub, rows_per_sub)],
                        out_hbm.at[pl.ds(s * rows_per_sub, rows_per_sub)])
        plsc.subcore_barrier()

        def body(j, carry):
            chunk = s + NS * j

            @pl.when(chunk < nchunks)
            def _():
                off = chunk * CH
                pltpu.sync_copy(rank_hbm.at[pl.ds(off, CH)], rank_v)
                pltpu.sync_copy(batch_hbm.at[pl.ds(off, CH)], bat_v)
                pltpu.sync_copy(h_hbm.at[pl.ds(off, CH)], rows_v)
                for v in range(CH // 16):
                    rv = rank_v[pl.ds(v * 16, 16)]
                    bv = bat_v[pl.ds(v * 16, 16)]
                    slot_v[pl.ds(v * 16, 16)] = jnp.where(
                        rv < k, bv * k + rv, nvalid + s)
                pltpu.async_copy(rows_v, out_hbm.at[slot_v], sem).wait()
            return carry

        lax.fori_loop(0, -(-nchunks // NS), body, 0)

    return pool


def kernel(x, edge_index, batch, W1_l, W1_r, b1, W2_l, W2_r, b2,
           W3_l, W3_r, b3, Wlin1, blin1, Wlin2, blin2):
    n, fin = x.shape
    e = edge_index.shape[1]
    h = W1_l.shape[1]
    num_graphs, k = 100, 30

    src = edge_index[0]
    dst = edge_index[1]
    zeros128 = jnp.zeros((n, W128), F32)
    ones128 = jnp.ones((CH, W128), F32)

    agg = _make_agg_kernel(n, e)
    cntk = _make_cnt_kernel(n, e)

    cnt_parts = cntk(dst, ones128, zeros128)
    c0, c1 = cnt_parts[:n], cnt_parts[n:]

    s_parts = agg(x, src, dst, zeros128)
    hcur = _layer_call(s_parts[:n], s_parts[n:], c0, c1, x,
                       W1_l, W1_r, b1.reshape(1, -1))
    s_parts = agg(hcur, src, dst, zeros128)
    hcur = _layer_call(s_parts[:n], s_parts[n:], c0, c1, hcur,
                       W2_l, W2_r, b2.reshape(1, -1))
    s_parts = agg(hcur, src, dst, zeros128)
    h3 = _layer_call(s_parts[:n], s_parts[n:], c0, c1, hcur,
                     W3_l, W3_r, b3.reshape(1, -1))

    key_col = h3[:, h - 1:h]
    rank = _rank_call(key_col, batch.reshape(n, 1),
                      key_col.reshape(1, n), batch.reshape(1, n))

    poolk = _make_pool_kernel(n, num_graphs, k)
    pooled_buf = poolk(h3, rank.reshape(n), batch, zeros128)
    pooled = pooled_buf[:num_graphs * k, :h].reshape(num_graphs, k * h)

    return _mlp_call(pooled, Wlin1, blin1.reshape(1, -1),
                     Wlin2, blin2.reshape(1, -1))
